# baseline (speedup 1.0000x reference)
import functools

import jax
import jax.numpy as jnp
from jax import lax
from jax.experimental import pallas as pl
from jax.experimental.pallas import tpu as pltpu
from jax.experimental.pallas import tpu_sc as plsc

WORD_DIM = 64
SHAPE_DIM = 16
OUT_DIM = WORD_DIM + SHAPE_DIM
PAIR_DIM = 2 * WORD_DIM

_NUM_CORES = 2
_NUM_SUBCORES = 16
_NUM_WORKERS = _NUM_CORES * _NUM_SUBCORES

_IDX_W = 128
_CHUNK = 256
_NIDX = _CHUNK // _IDX_W
_L = 16


def _make_body(batch, seq):
    blocks_per_seq = batch // _CHUNK
    num_chunks_total = seq * blocks_per_seq
    num_chunks = num_chunks_total // _NUM_WORKERS
    assert num_chunks % 2 == 0

    def body(whalf_hbm, p64_hbm, sidx_hbm, wtab_hbm, stab_hbm, out_hbm,
             wh0, wh1, p0, p1, s0, s1, pair0, pair1, comb_t, stab_v,
             g0, g1, wsem):
        wh = (wh0, wh1)
        pp = (p0, p1)
        ss = (s0, s1)
        pair = (pair0, pair1)
        gsem = (g0, g1)
        sid = lax.axis_index("s")
        wid = sid * _NUM_CORES + lax.axis_index("c")
        chunk0 = wid * num_chunks

        pltpu.sync_copy(stab_hbm, stab_v)
        iota = lax.iota(jnp.int32, _L)

        def out_dst(ci):
            s_pos = ci // blocks_per_seq
            blk = ci % blocks_per_seq
            return out_hbm.at[s_pos, :, pl.ds(blk * _CHUNK, _CHUNK)]

        def fire(ci, b):
            pltpu.sync_copy(whalf_hbm.at[ci], wh[b])
            pltpu.async_copy(p64_hbm.at[ci], pp[b], gsem[b])
            pltpu.async_copy(sidx_hbm.at[ci], ss[b], gsem[b])
            for j in range(_NIDX):
                pltpu.async_copy(
                    wtab_hbm.at[wh[b].at[j]],
                    pair[b].at[pl.ds(j * _IDX_W, _IDX_W)], gsem[b])

        def gather_wait(ci, b):
            pltpu.make_async_copy(p64_hbm.at[ci], pp[b], gsem[b]).wait()
            pltpu.make_async_copy(sidx_hbm.at[ci], ss[b], gsem[b]).wait()
            for j in range(_NIDX):
                pltpu.make_async_copy(
                    wtab_hbm.at[wh[b].at[j]],
                    pair[b].at[pl.ds(j * _IDX_W, _IDX_W)], gsem[b]).wait()

        def write_wait(ci):
            pltpu.make_async_copy(comb_t, out_dst(ci), wsem).wait()

        def select(b):
            def step(t, carry2):
                rowv = t * _L + iota
                src = pp[b][pl.ds(t * _L, _L)]
                sv = ss[b][pl.ds(t * _L, _L)] * SHAPE_DIM

                def cgroup(k, colw):
                    for _ in range(_L):
                        val = plsc.load_gather(pair[b], [rowv, src + colw])
                        plsc.store_scatter(comb_t, [colw, rowv], val)
                        colw = (colw + 1) & (WORD_DIM - 1)
                    return colw

                lax.fori_loop(0, WORD_DIM // _L, cgroup, iota)

                def sgroup(k, cols):
                    for _ in range(_L):
                        val = plsc.load_gather(stab_v, [sv + cols])
                        plsc.store_scatter(
                            comb_t, [WORD_DIM + cols, rowv], val)
                        cols = (cols + 1) & (SHAPE_DIM - 1)
                    return cols

                lax.fori_loop(0, 1, sgroup, iota & (SHAPE_DIM - 1))
                return carry2

            lax.fori_loop(0, _CHUNK // _L, step, 0)

        fire(chunk0, 0)

        def pair_iter(g, carry):
            ci0 = chunk0 + 2 * g
            for b in range(2):
                ci = ci0 + b
                nxt = ci + 1

                @pl.when(nxt < chunk0 + num_chunks)
                def _():
                    fire(nxt, 1 - b)

                gather_wait(ci, b)

                @pl.when(ci > chunk0)
                def _():
                    write_wait(ci - 1)

                select(b)
                pltpu.async_copy(comb_t, out_dst(ci), wsem)
            return carry

        lax.fori_loop(0, num_chunks // 2, pair_iter, 0)
        write_wait(chunk0 + num_chunks - 1)

    return body


@jax.jit
def kernel(word_id, shape_id, word_table, shape_table):
    b, s = word_id.shape
    num_chunks_total = (b * s) // _CHUNK
    wvocab = word_table.shape[0]
    svocab = shape_table.shape[0]

    wi_t = word_id.T.astype(jnp.int32)
    si_t = shape_id.T.astype(jnp.int32)
    whalf = (wi_t >> 1).reshape(num_chunks_total, _NIDX, _IDX_W)
    p64 = ((wi_t & 1) << 6).reshape(num_chunks_total, _CHUNK)
    sidx = si_t.reshape(num_chunks_total, _CHUNK)
    stab_flat = shape_table.reshape(svocab * SHAPE_DIM)
    wtab2 = word_table.reshape(wvocab // 2, PAIR_DIM)

    call = functools.partial(
        pl.kernel,
        out_type=jax.ShapeDtypeStruct((s, OUT_DIM, b), jnp.float32),
        mesh=plsc.VectorSubcoreMesh(core_axis_name="c", subcore_axis_name="s"),
        compiler_params=pltpu.CompilerParams(needs_layout_passes=False,
                                             disable_bounds_checks=True),
        scratch_types=[
            pltpu.VMEM((_NIDX, _IDX_W), jnp.int32),
            pltpu.VMEM((_NIDX, _IDX_W), jnp.int32),
            pltpu.VMEM((_CHUNK,), jnp.int32),
            pltpu.VMEM((_CHUNK,), jnp.int32),
            pltpu.VMEM((_CHUNK,), jnp.int32),
            pltpu.VMEM((_CHUNK,), jnp.int32),
            pltpu.VMEM((_CHUNK, PAIR_DIM), jnp.float32),
            pltpu.VMEM((_CHUNK, PAIR_DIM), jnp.float32),
            pltpu.VMEM((OUT_DIM, _CHUNK), jnp.float32),
            pltpu.VMEM((svocab * SHAPE_DIM,), jnp.float32),
            pltpu.SemaphoreType.DMA,
            pltpu.SemaphoreType.DMA,
            pltpu.SemaphoreType.DMA,
        ],
    )(_make_body(b, s))
    out_t = call(whalf, p64, sidx, wtab2, stab_flat)
    return jnp.transpose(out_t, (2, 0, 1))
